# baseline (speedup 1.0000x reference)
import jax
import jax.numpy as jnp
from jax import lax
from jax.experimental import pallas as pl
from jax.experimental.pallas import tpu as pltpu
from jax.experimental.pallas import tpu_sc as plsc

N = 10000
D = 128
H = 128
A = 64
B = 1024
E = 320000

NC, NS = 2, 16
NW = NC * NS
N_PAD = 10240
CH = 128
CHUNKS = 80
EW = CHUNKS * CH
E_PAD = NW * EW
RPT = N_PAD // NS
DEGW = 8

_mesh = plsc.VectorSubcoreMesh(core_axis_name="c", subcore_axis_name="s")


def _deg_body(dst_hbm, zeros_hbm, out_hbm, idx_v, ones_v, acc):
    c = lax.axis_index("c")
    s = lax.axis_index("s")
    wid = s * NC + c
    pltpu.sync_copy(zeros_hbm.at[pl.ds(s * RPT, RPT)], acc.at[pl.ds(s * RPT, RPT)])
    for k in range(CH // 16):
        ones_v[pl.ds(k * 16, 16)] = jnp.full((16,), 1.0, jnp.float32)
    plsc.subcore_barrier()
    base = wid * EW

    def body(i, carry):
        pltpu.sync_copy(dst_hbm.at[pl.ds(base + i * CH, CH)], idx_v)
        pltpu.sync_copy(ones_v, acc.at[idx_v], add=True)
        return carry

    lax.fori_loop(0, CHUNKS, body, 0)
    plsc.subcore_barrier()
    pltpu.sync_copy(acc.at[pl.ds(s * RPT, RPT)],
                    out_hbm.at[pl.ds(c * N_PAD + s * RPT, RPT)])


_deg_call = pl.kernel(
    _deg_body,
    out_type=jax.ShapeDtypeStruct((NC * N_PAD,), jnp.float32),
    mesh=_mesh,
    scratch_types=[
        pltpu.VMEM((CH,), jnp.int32),
        pltpu.VMEM((CH,), jnp.float32),
        pltpu.VMEM_SHARED((N_PAD,), jnp.float32),
    ],
)


def _mp_body(g_hbm, src_hbm, dst_hbm, zeros_hbm, out_hbm,
             dst_slab, r0, r1, sv0, sv1, dv0, dv1, acc,
             gs0, gs1, ss0, ss1, is0, is1):
    rows = [r0, r1]
    srcv = [sv0, sv1]
    dstv = [dv0, dv1]
    gsem = [gs0, gs1]
    ssem = [ss0, ss1]
    isem = [is0, is1]
    c = lax.axis_index("c")
    s = lax.axis_index("s")
    wid = s * NC + c
    base = wid * EW

    pltpu.sync_copy(zeros_hbm.at[pl.ds(s * RPT, RPT)], acc.at[pl.ds(s * RPT, RPT)])
    pltpu.sync_copy(dst_hbm.at[pl.ds(base, EW)], dst_slab)
    pltpu.sync_copy(src_hbm.at[pl.ds(base, CH)], sv0)
    plsc.subcore_barrier()

    def fire_idx(j, b):
        pltpu.async_copy(src_hbm.at[pl.ds(base + j * CH, CH)], srcv[b], isem[b])

    def wait_idx(b):
        pltpu.make_async_copy(src_hbm.at[pl.ds(base, CH)], srcv[b], isem[b]).wait()

    def fire_gather(b):
        pltpu.async_copy(g_hbm.at[srcv[b]], rows[b], gsem[b])

    def wait_gather(b):
        pltpu.make_async_copy(g_hbm.at[srcv[b]], rows[b], gsem[b]).wait()

    def load_dstv(j, b):
        for k in range(CH // 16):
            dstv[b][pl.ds(k * 16, 16)] = dst_slab[pl.ds(j * CH + k * 16, 16)]

    def fire_scatter(b):
        pltpu.async_copy(rows[b], acc.at[dstv[b]], ssem[b], add=True)

    def wait_scatter(b):
        pltpu.make_async_copy(rows[b], acc.at[dstv[b]], ssem[b]).wait()

    fire_gather(0)
    fire_idx(1, 1)

    wait_idx(1)
    fire_gather(1)
    wait_gather(0)
    fire_idx(2, 0)
    load_dstv(0, 0)
    fire_scatter(0)

    def steady(i, carry):
        for t in range(2):
            j = 2 * i + 2 + t
            b = t
            bb = 1 - b
            wait_scatter(b)
            wait_idx(b)
            fire_gather(b)
            wait_gather(bb)
            fire_idx(j + 1, bb)
            load_dstv(j - 1, bb)
            fire_scatter(bb)
        return carry

    lax.fori_loop(0, (CHUNKS - 4) // 2, steady, 0)

    wait_scatter(0)
    wait_idx(0)
    fire_gather(0)
    wait_gather(1)
    fire_idx(CHUNKS - 1, 1)
    load_dstv(CHUNKS - 3, 1)
    fire_scatter(1)

    wait_scatter(1)
    wait_idx(1)
    fire_gather(1)
    wait_gather(0)
    load_dstv(CHUNKS - 2, 0)
    fire_scatter(0)

    wait_gather(1)
    load_dstv(CHUNKS - 1, 1)
    fire_scatter(1)
    wait_scatter(0)
    wait_scatter(1)

    plsc.subcore_barrier()
    pltpu.sync_copy(acc.at[pl.ds(s * RPT, RPT)], out_hbm.at[c, pl.ds(s * RPT, RPT)])


_mp_call = pl.kernel(
    _mp_body,
    out_type=jax.ShapeDtypeStruct((NC, N_PAD, H), jnp.float32),
    mesh=_mesh,
    scratch_types=[
        pltpu.VMEM((EW,), jnp.int32),
        pltpu.VMEM((CH, H), jnp.float32),
        pltpu.VMEM((CH, H), jnp.float32),
        pltpu.VMEM((CH,), jnp.int32),
        pltpu.VMEM((CH,), jnp.int32),
        pltpu.VMEM((CH,), jnp.int32),
        pltpu.VMEM((CH,), jnp.int32),
        pltpu.VMEM_SHARED((N_PAD, H), jnp.float32),
        pltpu.SemaphoreType.DMA,
        pltpu.SemaphoreType.DMA,
        pltpu.SemaphoreType.DMA,
        pltpu.SemaphoreType.DMA,
        pltpu.SemaphoreType.DMA,
        pltpu.SemaphoreType.DMA,
    ],
)


BN = 1024
GN = N_PAD // BN


def _dinv_of(deg_blk):
    d = deg_blk[:, 0:1] + deg_blk[:, 1:2]
    return lax.rsqrt(d + 1.0)


def _l1_body(deg_ref, x_ref, w1_ref, g1_ref):
    dinv = _dinv_of(deg_ref)
    g = lax.dot_general(x_ref[...], w1_ref[...], (((0,), (0,)), ((), ())),
                        preferred_element_type=jnp.float32)
    g1_ref[...] = g * dinv


_l1_call = pl.pallas_call(
    _l1_body,
    grid=(GN,),
    in_specs=[
        pl.BlockSpec((BN, NC), lambda i: (i, 0)),
        pl.BlockSpec((D, BN), lambda i: (0, i)),
        pl.BlockSpec((D, H), lambda i: (0, 0)),
    ],
    out_specs=pl.BlockSpec((BN, H), lambda i: (i, 0)),
    out_shape=jax.ShapeDtypeStruct((N_PAD, H), jnp.float32),
)


def _l2_body(deg_ref, s1_ref, g1_ref, w2_ref, b1_ref, g2_ref):
    dinv = _dinv_of(deg_ref)
    h1 = jnp.maximum(dinv * (s1_ref[0] + s1_ref[1] + g1_ref[...]) + b1_ref[...], 0.0)
    g2_ref[...] = dinv * jnp.dot(h1, w2_ref[...], preferred_element_type=jnp.float32)


_l2_call = pl.pallas_call(
    _l2_body,
    grid=(GN,),
    in_specs=[
        pl.BlockSpec((BN, NC), lambda i: (i, 0)),
        pl.BlockSpec((NC, BN, H), lambda i: (0, i, 0)),
        pl.BlockSpec((BN, H), lambda i: (i, 0)),
        pl.BlockSpec((H, H), lambda i: (0, 0)),
        pl.BlockSpec((1, H), lambda i: (0, 0)),
    ],
    out_specs=pl.BlockSpec((BN, H), lambda i: (i, 0)),
    out_shape=jax.ShapeDtypeStruct((N_PAD, H), jnp.float32),
)


def _l3_body(deg_ref, s2_ref, g2_ref, b2_ref, h2_ref):
    dinv = _dinv_of(deg_ref)
    h2_ref[...] = dinv * (s2_ref[0] + s2_ref[1] + g2_ref[...]) + b2_ref[...]


_l3_call = pl.pallas_call(
    _l3_body,
    grid=(GN,),
    in_specs=[
        pl.BlockSpec((BN, NC), lambda i: (i, 0)),
        pl.BlockSpec((NC, BN, H), lambda i: (0, i, 0)),
        pl.BlockSpec((BN, H), lambda i: (i, 0)),
        pl.BlockSpec((1, H), lambda i: (0, 0)),
    ],
    out_specs=pl.BlockSpec((BN, H), lambda i: (i, 0)),
    out_shape=jax.ShapeDtypeStruct((N_PAD, H), jnp.float32),
)


BB = 256


def _out_body(o_ref, h2_ref, out_ref):
    out_ref[...] = jnp.dot(o_ref[...], h2_ref[...], preferred_element_type=jnp.float32)


_out_call = pl.pallas_call(
    _out_body,
    grid=(B // BB,),
    in_specs=[
        pl.BlockSpec((BB, N), lambda i: (i, 0)),
        pl.BlockSpec((N, A), lambda i: (0, 0)),
    ],
    out_specs=pl.BlockSpec((BB, A), lambda i: (i, 0)),
    out_shape=jax.ShapeDtypeStruct((B, A), jnp.float32),
)


def kernel(x, edge_index, onehot_values, W1, b1, W2, b2):
    ei = edge_index.astype(jnp.int32)
    pad = jnp.full((E_PAD - E,), N, dtype=jnp.int32)
    src = jnp.concatenate([ei[0], pad])
    dst = jnp.concatenate([ei[1], pad])
    xp = jnp.pad(x, ((0, 0), (0, N_PAD - N)))
    w2p = jnp.pad(W2, ((0, 0), (0, H - A)))
    b1r = b1.reshape(1, H)
    b2r = jnp.pad(b2, (0, H - A)).reshape(1, H)
    zeros_1 = jnp.zeros((N_PAD,), jnp.float32)
    zeros_c = jnp.zeros((N_PAD, H), jnp.float32)

    degp = _deg_call(dst, zeros_1)
    deg = degp.reshape(NC, N_PAD).T
    g1 = _l1_call(deg, xp, W1)
    s1 = _mp_call(g1, src, dst, zeros_c)
    g2 = _l2_call(deg, s1, g1, w2p, b1r)
    s2 = _mp_call(g2, src, dst, zeros_c)
    h2 = _l3_call(deg, s2, g2, b2r)
    return _out_call(onehot_values, h2[:N, :A])


# speedup vs baseline: 3.3625x; 3.3625x over previous
import jax
import jax.numpy as jnp
from jax import lax
from jax.experimental import pallas as pl
from jax.experimental.pallas import tpu as pltpu
from jax.experimental.pallas import tpu_sc as plsc

N = 10000
D = 128
H = 128
A = 64
B = 1024
E = 320000

NC, NS = 2, 16
NW = NC * NS
N_PAD = 10240
CH = 128
CHUNKS = 80
EW = CHUNKS * CH
E_PAD = NW * EW
RPT = N_PAD // NS
DEGW = 8

_mesh = plsc.VectorSubcoreMesh(core_axis_name="c", subcore_axis_name="s")


def _deg_body(dst_hbm, zeros_hbm, out_hbm, idx_v, ones_v, acc):
    c = lax.axis_index("c")
    s = lax.axis_index("s")
    wid = s * NC + c
    pltpu.sync_copy(zeros_hbm.at[pl.ds(s * RPT, RPT)], acc.at[pl.ds(s * RPT, RPT)])
    for k in range(CH // 16):
        ones_v[pl.ds(k * 16, 16)] = jnp.full((16,), 1.0, jnp.float32)
    plsc.subcore_barrier()
    base = wid * EW

    def body(i, carry):
        pltpu.sync_copy(dst_hbm.at[pl.ds(base + i * CH, CH)], idx_v)
        pltpu.sync_copy(ones_v, acc.at[idx_v], add=True)
        return carry

    lax.fori_loop(0, CHUNKS, body, 0)
    plsc.subcore_barrier()
    pltpu.sync_copy(acc.at[pl.ds(s * RPT, RPT)],
                    out_hbm.at[pl.ds(c * N_PAD + s * RPT, RPT)])


_deg_call = pl.kernel(
    _deg_body,
    out_type=jax.ShapeDtypeStruct((NC * N_PAD,), jnp.float32),
    mesh=_mesh,
    scratch_types=[
        pltpu.VMEM((CH,), jnp.int32),
        pltpu.VMEM((CH,), jnp.float32),
        pltpu.VMEM_SHARED((N_PAD,), jnp.float32),
    ],
)


def _mp_body(g_hbm, src_hbm, dst_hbm, zeros_hbm, out_hbm,
             dst_slab, r0, r1, sv0, sv1, dv0, dv1, acc,
             gs0, gs1, ss0, ss1, is0, is1):
    rows = [r0, r1]
    srcv = [sv0, sv1]
    dstv = [dv0, dv1]
    gsem = [gs0, gs1]
    ssem = [ss0, ss1]
    isem = [is0, is1]
    c = lax.axis_index("c")
    s = lax.axis_index("s")
    wid = s * NC + c
    base = wid * EW

    pltpu.sync_copy(zeros_hbm.at[pl.ds(s * RPT, RPT)], acc.at[pl.ds(s * RPT, RPT)])
    pltpu.sync_copy(dst_hbm.at[pl.ds(base, EW)], dst_slab)
    pltpu.sync_copy(src_hbm.at[pl.ds(base, CH)], sv0)
    plsc.subcore_barrier()

    def fire_idx(j, b):
        pltpu.async_copy(src_hbm.at[pl.ds(base + j * CH, CH)], srcv[b], isem[b])

    def wait_idx(b):
        pltpu.make_async_copy(src_hbm.at[pl.ds(base, CH)], srcv[b], isem[b]).wait()

    def fire_gather(b):
        pltpu.async_copy(g_hbm.at[srcv[b]], rows[b], gsem[b])

    def wait_gather(b):
        pltpu.make_async_copy(g_hbm.at[srcv[b]], rows[b], gsem[b]).wait()

    def load_dstv(j, b):
        for k in range(CH // 16):
            dstv[b][pl.ds(k * 16, 16)] = dst_slab[pl.ds(j * CH + k * 16, 16)]

    def fire_scatter(b):
        pltpu.async_copy(rows[b], acc.at[dstv[b]], ssem[b], add=True)

    def wait_scatter(b):
        pltpu.make_async_copy(rows[b], acc.at[dstv[b]], ssem[b]).wait()

    fire_gather(0)
    fire_idx(1, 1)

    wait_idx(1)
    fire_gather(1)
    wait_gather(0)
    fire_idx(2, 0)
    load_dstv(0, 0)
    fire_scatter(0)

    def steady(i, carry):
        for t in range(2):
            j = 2 * i + 2 + t
            b = t
            bb = 1 - b
            wait_scatter(b)
            wait_idx(b)
            fire_gather(b)
            wait_gather(bb)
            fire_idx(j + 1, bb)
            load_dstv(j - 1, bb)
            fire_scatter(bb)
        return carry

    lax.fori_loop(0, (CHUNKS - 4) // 2, steady, 0)

    wait_scatter(0)
    wait_idx(0)
    fire_gather(0)
    wait_gather(1)
    fire_idx(CHUNKS - 1, 1)
    load_dstv(CHUNKS - 3, 1)
    fire_scatter(1)

    wait_scatter(1)
    wait_idx(1)
    fire_gather(1)
    wait_gather(0)
    load_dstv(CHUNKS - 2, 0)
    fire_scatter(0)

    wait_gather(1)
    load_dstv(CHUNKS - 1, 1)
    fire_scatter(1)
    wait_scatter(0)
    wait_scatter(1)

    plsc.subcore_barrier()
    pltpu.sync_copy(acc.at[pl.ds(s * RPT, RPT)], out_hbm.at[c, pl.ds(s * RPT, RPT)])


_mp_call = pl.kernel(
    _mp_body,
    out_type=jax.ShapeDtypeStruct((NC, N_PAD, H), jnp.float32),
    mesh=_mesh,
    scratch_types=[
        pltpu.VMEM((EW,), jnp.int32),
        pltpu.VMEM((CH, H), jnp.float32),
        pltpu.VMEM((CH, H), jnp.float32),
        pltpu.VMEM((CH,), jnp.int32),
        pltpu.VMEM((CH,), jnp.int32),
        pltpu.VMEM((CH,), jnp.int32),
        pltpu.VMEM((CH,), jnp.int32),
        pltpu.VMEM_SHARED((N_PAD, H), jnp.float32),
        pltpu.SemaphoreType.DMA,
        pltpu.SemaphoreType.DMA,
        pltpu.SemaphoreType.DMA,
        pltpu.SemaphoreType.DMA,
        pltpu.SemaphoreType.DMA,
        pltpu.SemaphoreType.DMA,
    ],
)


BN = 1024
GN = N_PAD // BN


def _dinv_of(deg_blk):
    d = deg_blk[:, 0:1] + deg_blk[:, 1:2]
    return lax.rsqrt(d + 1.0)


def _l1_body(deg_ref, x_ref, w1_ref, g1_ref):
    dinv = _dinv_of(deg_ref)
    g = lax.dot_general(x_ref[...], w1_ref[...], (((0,), (0,)), ((), ())),
                        preferred_element_type=jnp.float32)
    g1_ref[...] = g * dinv


_l1_call = pl.pallas_call(
    _l1_body,
    grid=(GN,),
    in_specs=[
        pl.BlockSpec((BN, NC), lambda i: (i, 0)),
        pl.BlockSpec((D, BN), lambda i: (0, i)),
        pl.BlockSpec((D, H), lambda i: (0, 0)),
    ],
    out_specs=pl.BlockSpec((BN, H), lambda i: (i, 0)),
    out_shape=jax.ShapeDtypeStruct((N_PAD, H), jnp.float32),
)


def _l2_body(deg_ref, s1_ref, g1_ref, w2_ref, b1_ref, g2_ref):
    dinv = _dinv_of(deg_ref)
    h1 = jnp.maximum(dinv * (s1_ref[0] + s1_ref[1] + g1_ref[...]) + b1_ref[...], 0.0)
    g2_ref[...] = dinv * jnp.dot(h1, w2_ref[...], preferred_element_type=jnp.float32)


_l2_call = pl.pallas_call(
    _l2_body,
    grid=(GN,),
    in_specs=[
        pl.BlockSpec((BN, NC), lambda i: (i, 0)),
        pl.BlockSpec((NC, BN, H), lambda i: (0, i, 0)),
        pl.BlockSpec((BN, H), lambda i: (i, 0)),
        pl.BlockSpec((H, H), lambda i: (0, 0)),
        pl.BlockSpec((1, H), lambda i: (0, 0)),
    ],
    out_specs=pl.BlockSpec((BN, H), lambda i: (i, 0)),
    out_shape=jax.ShapeDtypeStruct((N_PAD, H), jnp.float32),
)


def _l3_body(deg_ref, s2_ref, g2_ref, b2_ref, h2_ref):
    dinv = _dinv_of(deg_ref)
    h2_ref[...] = dinv * (s2_ref[0] + s2_ref[1] + g2_ref[...]) + b2_ref[...]


_l3_call = pl.pallas_call(
    _l3_body,
    grid=(GN,),
    in_specs=[
        pl.BlockSpec((BN, NC), lambda i: (i, 0)),
        pl.BlockSpec((NC, BN, H), lambda i: (0, i, 0)),
        pl.BlockSpec((BN, H), lambda i: (i, 0)),
        pl.BlockSpec((1, H), lambda i: (0, 0)),
    ],
    out_specs=pl.BlockSpec((BN, H), lambda i: (i, 0)),
    out_shape=jax.ShapeDtypeStruct((N_PAD, H), jnp.float32),
)


BB = 256


def _out_body(o_ref, h2_ref, out_ref):
    out_ref[...] = jnp.dot(o_ref[...], h2_ref[...], preferred_element_type=jnp.float32)


_out_call = pl.pallas_call(
    _out_body,
    grid=(B // BB,),
    in_specs=[
        pl.BlockSpec((BB, N), lambda i: (i, 0)),
        pl.BlockSpec((N, A), lambda i: (0, 0)),
    ],
    out_specs=pl.BlockSpec((BB, A), lambda i: (i, 0)),
    out_shape=jax.ShapeDtypeStruct((B, A), jnp.float32),
)


def kernel(x, edge_index, onehot_values, W1, b1, W2, b2):
    ei = edge_index.astype(jnp.int32)
    pad = N + (jnp.arange(E_PAD - E, dtype=jnp.int32) % (N_PAD - N))
    src = jnp.concatenate([ei[0], pad])
    dst = jnp.concatenate([ei[1], pad])
    xp = jnp.pad(x, ((0, 0), (0, N_PAD - N)))
    w2p = jnp.pad(W2, ((0, 0), (0, H - A)))
    b1r = b1.reshape(1, H)
    b2r = jnp.pad(b2, (0, H - A)).reshape(1, H)
    zeros_1 = jnp.zeros((N_PAD,), jnp.float32)
    zeros_c = jnp.zeros((N_PAD, H), jnp.float32)

    degp = _deg_call(dst, zeros_1)
    deg = degp.reshape(NC, N_PAD).T
    g1 = _l1_call(deg, xp, W1)
    s1 = _mp_call(g1, src, dst, zeros_c)
    g2 = _l2_call(deg, s1, g1, w2p, b1r)
    s2 = _mp_call(g2, src, dst, zeros_c)
    h2 = _l3_call(deg, s2, g2, b2r)
    return _out_call(onehot_values, h2[:N, :A])


# speedup vs baseline: 3.5601x; 1.0587x over previous
import jax
import jax.numpy as jnp
from jax import lax
from jax.experimental import pallas as pl
from jax.experimental.pallas import tpu as pltpu
from jax.experimental.pallas import tpu_sc as plsc

N = 10000
D = 128
H = 128
A = 64
B = 1024
E = 320000

NC, NS = 2, 16
NW = NC * NS
N_PAD = 10240
CH = 128
CHUNKS = 80
EW = CHUNKS * CH
E_PAD = NW * EW
RPT = N_PAD // NS
N_MP = 10112
RPT_MP = N_MP // NS
DEGW = 8

_mesh = plsc.VectorSubcoreMesh(core_axis_name="c", subcore_axis_name="s")


def _deg_body(dst_hbm, zeros_hbm, out_hbm, idx_v, ones_v, acc):
    c = lax.axis_index("c")
    s = lax.axis_index("s")
    wid = s * NC + c
    pltpu.sync_copy(zeros_hbm.at[pl.ds(s * RPT, RPT)], acc.at[pl.ds(s * RPT, RPT)])
    for k in range(CH // 16):
        ones_v[pl.ds(k * 16, 16)] = jnp.full((16,), 1.0, jnp.float32)
    plsc.subcore_barrier()
    base = wid * EW

    def body(i, carry):
        pltpu.sync_copy(dst_hbm.at[pl.ds(base + i * CH, CH)], idx_v)
        pltpu.sync_copy(ones_v, acc.at[idx_v], add=True)
        return carry

    lax.fori_loop(0, CHUNKS, body, 0)
    plsc.subcore_barrier()
    pltpu.sync_copy(acc.at[pl.ds(s * RPT, RPT)],
                    out_hbm.at[pl.ds(c * N_PAD + s * RPT, RPT)])


_deg_call = pl.kernel(
    _deg_body,
    out_type=jax.ShapeDtypeStruct((NC * N_PAD,), jnp.float32),
    mesh=_mesh,
    scratch_types=[
        pltpu.VMEM((CH,), jnp.int32),
        pltpu.VMEM((CH,), jnp.float32),
        pltpu.VMEM_SHARED((N_PAD,), jnp.float32),
    ],
)


def _mp_body(g_hbm, src_hbm, dst_hbm, zeros_hbm, out_hbm,
             r0, r1, r2, sv0, sv1, sv2, sv3, dv0, dv1, dv2, acc,
             gS0, gS1, gS2, sS0, sS1, sS2, iS0, iS1, iS2, iS3, dS0, dS1, dS2):
    rows = [r0, r1, r2]
    srcv = [sv0, sv1, sv2, sv3]
    dstv = [dv0, dv1, dv2]
    gsem = [gS0, gS1, gS2]
    ssem = [sS0, sS1, sS2]
    isem = [iS0, iS1, iS2, iS3]
    dsem = [dS0, dS1, dS2]
    c = lax.axis_index("c")
    s = lax.axis_index("s")
    wid = s * NC + c
    base = wid * EW

    pltpu.sync_copy(zeros_hbm.at[pl.ds(s * RPT_MP, RPT_MP)],
                    acc.at[pl.ds(s * RPT_MP, RPT_MP)])
    pltpu.sync_copy(src_hbm.at[pl.ds(base, CH)], sv0)
    for m in (1, 2, 3):
        pltpu.async_copy(src_hbm.at[pl.ds(base + m * CH, CH)], srcv[m], isem[m])
    for m in (0, 1, 2):
        pltpu.sync_copy(dst_hbm.at[pl.ds(base + m * CH, CH)], dstv[m])
    plsc.subcore_barrier()

    def w_scatter(m):
        q = m % 3
        pltpu.make_async_copy(rows[q], acc.at[dstv[q]], ssem[q]).wait()

    def f_dst(m):
        q = m % 3
        pltpu.async_copy(dst_hbm.at[pl.ds(base + m * CH, CH)], dstv[q], dsem[q])

    def w_gather(m):
        q = m % 3
        pltpu.make_async_copy(g_hbm.at[srcv[m % 4]], rows[q], gsem[q]).wait()

    def w_dst(m):
        q = m % 3
        pltpu.make_async_copy(dst_hbm.at[pl.ds(base, CH)], dstv[q], dsem[q]).wait()

    def f_scatter(m):
        q = m % 3
        pltpu.async_copy(rows[q], acc.at[dstv[q]], ssem[q], add=True)

    def f_src(m):
        q = m % 4
        pltpu.async_copy(src_hbm.at[pl.ds(base + m * CH, CH)], srcv[q], isem[q])

    def w_src(m):
        q = m % 4
        pltpu.make_async_copy(src_hbm.at[pl.ds(base, CH)], srcv[q], isem[q]).wait()

    def f_gather(m):
        q = m % 3
        pltpu.async_copy(g_hbm.at[srcv[m % 4]], rows[q], gsem[q])

    def step(j):
        if 3 <= j and j - 3 < CHUNKS:
            w_scatter(j - 3)
        if 3 <= j < CHUNKS:
            f_dst(j)
        if 2 <= j and j - 2 < CHUNKS:
            w_gather(j - 2)
            if j >= 5:
                w_dst(j - 2)
            f_scatter(j - 2)
        if 2 <= j and j + 2 < CHUNKS:
            f_src(j + 2)
        if j < CHUNKS:
            if j >= 1:
                w_src(j)
            f_gather(j)

    for j in range(6):
        step(j)

    def steady(i, carry):
        for t in range(12):
            step_steady(6 + 12 * i + t, t)
        return carry

    def step_steady(j, t):
        jj = 6 + t
        q3, p3, r3 = jj % 3, (jj - 3) % 3, (jj - 2) % 3
        q4, p4 = jj % 4, (jj + 2) % 4
        pltpu.make_async_copy(rows[p3], acc.at[dstv[p3]], ssem[p3]).wait()
        pltpu.async_copy(dst_hbm.at[pl.ds(base + j * CH, CH)], dstv[p3], dsem[p3])
        pltpu.make_async_copy(g_hbm.at[srcv[(jj - 2) % 4]], rows[r3], gsem[r3]).wait()
        pltpu.make_async_copy(dst_hbm.at[pl.ds(base, CH)], dstv[r3], dsem[r3]).wait()
        pltpu.async_copy(rows[r3], acc.at[dstv[r3]], ssem[r3], add=True)
        pltpu.async_copy(src_hbm.at[pl.ds(base + (j + 2) * CH, CH)], srcv[p4], isem[p4])
        pltpu.make_async_copy(src_hbm.at[pl.ds(base, CH)], srcv[q4], isem[q4]).wait()
        pltpu.async_copy(g_hbm.at[srcv[q4]], rows[q3], gsem[q3])

    lax.fori_loop(0, (CHUNKS - 8) // 12, steady, 0)

    for j in range(CHUNKS - 2, CHUNKS + 3):
        step(j)

    plsc.subcore_barrier()
    pltpu.sync_copy(acc.at[pl.ds(s * RPT_MP, RPT_MP)],
                    out_hbm.at[c, pl.ds(s * RPT_MP, RPT_MP)])


_mp_call = pl.kernel(
    _mp_body,
    out_type=jax.ShapeDtypeStruct((NC, N_MP, H), jnp.float32),
    mesh=_mesh,
    scratch_types=(
        [pltpu.VMEM((CH, H), jnp.float32)] * 3
        + [pltpu.VMEM((CH,), jnp.int32)] * 7
        + [pltpu.VMEM_SHARED((N_MP, H), jnp.float32)]
        + [pltpu.SemaphoreType.DMA] * 13
    ),
)


BN = 1024
GN = N_PAD // BN


def _dinv_of(deg_blk):
    d = deg_blk[:, 0:1] + deg_blk[:, 1:2]
    return lax.rsqrt(d + 1.0)


def _l1_body(deg_ref, x_ref, w1_ref, g1_ref):
    dinv = _dinv_of(deg_ref)
    g = lax.dot_general(x_ref[...], w1_ref[...], (((0,), (0,)), ((), ())),
                        preferred_element_type=jnp.float32)
    g1_ref[...] = g * dinv


_l1_call = pl.pallas_call(
    _l1_body,
    grid=(GN,),
    in_specs=[
        pl.BlockSpec((BN, NC), lambda i: (i, 0)),
        pl.BlockSpec((D, BN), lambda i: (0, i)),
        pl.BlockSpec((D, H), lambda i: (0, 0)),
    ],
    out_specs=pl.BlockSpec((BN, H), lambda i: (i, 0)),
    out_shape=jax.ShapeDtypeStruct((N_PAD, H), jnp.float32),
)


def _l2_body(deg_ref, s1_ref, g1_ref, w2_ref, b1_ref, g2_ref):
    dinv = _dinv_of(deg_ref)
    h1 = jnp.maximum(dinv * (s1_ref[0] + s1_ref[1] + g1_ref[...]) + b1_ref[...], 0.0)
    g2_ref[...] = dinv * jnp.dot(h1, w2_ref[...], preferred_element_type=jnp.float32)


_l2_call = pl.pallas_call(
    _l2_body,
    grid=(GN,),
    in_specs=[
        pl.BlockSpec((BN, NC), lambda i: (i, 0)),
        pl.BlockSpec((NC, BN, H), lambda i: (0, i, 0)),
        pl.BlockSpec((BN, H), lambda i: (i, 0)),
        pl.BlockSpec((H, H), lambda i: (0, 0)),
        pl.BlockSpec((1, H), lambda i: (0, 0)),
    ],
    out_specs=pl.BlockSpec((BN, H), lambda i: (i, 0)),
    out_shape=jax.ShapeDtypeStruct((N_PAD, H), jnp.float32),
)


def _l3_body(deg_ref, s2_ref, g2_ref, b2_ref, h2_ref):
    dinv = _dinv_of(deg_ref)
    h2_ref[...] = dinv * (s2_ref[0] + s2_ref[1] + g2_ref[...]) + b2_ref[...]


_l3_call = pl.pallas_call(
    _l3_body,
    grid=(GN,),
    in_specs=[
        pl.BlockSpec((BN, NC), lambda i: (i, 0)),
        pl.BlockSpec((NC, BN, H), lambda i: (0, i, 0)),
        pl.BlockSpec((BN, H), lambda i: (i, 0)),
        pl.BlockSpec((1, H), lambda i: (0, 0)),
    ],
    out_specs=pl.BlockSpec((BN, H), lambda i: (i, 0)),
    out_shape=jax.ShapeDtypeStruct((N_PAD, H), jnp.float32),
)


BB = 256


def _out_body(o_ref, h2_ref, out_ref):
    out_ref[...] = jnp.dot(o_ref[...], h2_ref[...], preferred_element_type=jnp.float32)


_out_call = pl.pallas_call(
    _out_body,
    grid=(B // BB,),
    in_specs=[
        pl.BlockSpec((BB, N), lambda i: (i, 0)),
        pl.BlockSpec((N, A), lambda i: (0, 0)),
    ],
    out_specs=pl.BlockSpec((BB, A), lambda i: (i, 0)),
    out_shape=jax.ShapeDtypeStruct((B, A), jnp.float32),
)


def kernel(x, edge_index, onehot_values, W1, b1, W2, b2):
    ei = edge_index.astype(jnp.int32)
    pad = N + (jnp.arange(E_PAD - E, dtype=jnp.int32) % (N_MP - N))
    src = jnp.concatenate([ei[0], pad])
    dst = jnp.concatenate([ei[1], pad])
    xp = jnp.pad(x, ((0, 0), (0, N_PAD - N)))
    w2p = jnp.pad(W2, ((0, 0), (0, H - A)))
    b1r = b1.reshape(1, H)
    b2r = jnp.pad(b2, (0, H - A)).reshape(1, H)
    zeros_1 = jnp.zeros((N_PAD,), jnp.float32)
    zeros_c = jnp.zeros((N_PAD, H), jnp.float32)

    degp = _deg_call(dst, zeros_1)
    deg = degp.reshape(NC, N_PAD).T
    g1 = _l1_call(deg, xp, W1)
    s1 = _mp_call(g1, src, dst, zeros_c)
    g2 = _l2_call(deg, s1, g1, w2p, b1r)
    s2 = _mp_call(g2, src, dst, zeros_c)
    h2 = _l3_call(deg, s2, g2, b2r)
    return _out_call(onehot_values, h2[:N, :A])


# speedup vs baseline: 3.9066x; 1.0973x over previous
import jax
import jax.numpy as jnp
from jax import lax
from jax.experimental import pallas as pl
from jax.experimental.pallas import tpu as pltpu
from jax.experimental.pallas import tpu_sc as plsc

N = 10000
D = 128
H = 128
A = 64
B = 1024
E = 320000

NC, NS = 2, 16
NW = NC * NS
N_PAD = 10240
CH = 128
CHUNKS = 80
EW = CHUNKS * CH
E_PAD = NW * EW
RPT = N_PAD // NS
N_MP = 10112
RPT_MP = N_MP // NS
DEGW = 8

_mesh = plsc.VectorSubcoreMesh(core_axis_name="c", subcore_axis_name="s")


def _deg_body(dst_hbm, zeros_hbm, out_hbm, iv0, iv1, iv2, iv3, ones_v, acc,
              sS0, sS1, sS2, sS3, iS0, iS1, iS2, iS3):
    idxv = [iv0, iv1, iv2, iv3]
    ssem = [sS0, sS1, sS2, sS3]
    isem = [iS0, iS1, iS2, iS3]
    c = lax.axis_index("c")
    s = lax.axis_index("s")
    wid = s * NC + c
    base = wid * EW
    pltpu.sync_copy(zeros_hbm.at[pl.ds(s * RPT, RPT)], acc.at[pl.ds(s * RPT, RPT)])
    for k in range(CH // 16):
        ones_v[pl.ds(k * 16, 16)] = jnp.full((16,), 1.0, jnp.float32)
    pltpu.sync_copy(dst_hbm.at[pl.ds(base, CH)], iv0)
    for m in (1, 2, 3):
        pltpu.async_copy(dst_hbm.at[pl.ds(base + m * CH, CH)], idxv[m], isem[m])
    plsc.subcore_barrier()

    def step(j, static_q, steady_mode=False):
        q = static_q % 4
        if steady_mode or j >= 2:
            qq = (static_q - 2) % 4
            pltpu.make_async_copy(ones_v, acc.at[idxv[qq]], ssem[qq]).wait()
            if steady_mode or j + 2 < CHUNKS:
                pltpu.async_copy(dst_hbm.at[pl.ds(base + (j + 2) * CH, CH)],
                                 idxv[qq], isem[qq])
        if steady_mode or j < CHUNKS:
            if steady_mode or j >= 1:
                pltpu.make_async_copy(dst_hbm.at[pl.ds(base, CH)], idxv[q],
                                      isem[q]).wait()
            pltpu.async_copy(ones_v, acc.at[idxv[q]], ssem[q], add=True)

    for j in range(4):
        step(j, j)

    def steady(i, carry):
        for t in range(4):
            step(4 + 4 * i + t, t, steady_mode=True)
        return carry

    lax.fori_loop(0, (CHUNKS - 8) // 4, steady, 0)

    for j in range(CHUNKS - 4, CHUNKS + 2):
        step(j, j)

    plsc.subcore_barrier()
    pltpu.sync_copy(acc.at[pl.ds(s * RPT, RPT)],
                    out_hbm.at[pl.ds(c * N_PAD + s * RPT, RPT)])


_deg_call = pl.kernel(
    _deg_body,
    out_type=jax.ShapeDtypeStruct((NC * N_PAD,), jnp.float32),
    mesh=_mesh,
    scratch_types=(
        [pltpu.VMEM((CH,), jnp.int32)] * 4
        + [pltpu.VMEM((CH,), jnp.float32)]
        + [pltpu.VMEM_SHARED((N_PAD,), jnp.float32)]
        + [pltpu.SemaphoreType.DMA] * 8
    ),
)


def _mp_body(g_hbm, src_hbm, dst_hbm, zeros_hbm, out_hbm,
             r0, r1, r2, sv0, sv1, sv2, sv3, dv0, dv1, dv2, acc,
             gS0, gS1, gS2, sS0, sS1, sS2, iS0, iS1, iS2, iS3, dS0, dS1, dS2):
    rows = [r0, r1, r2]
    srcv = [sv0, sv1, sv2, sv3]
    dstv = [dv0, dv1, dv2]
    gsem = [gS0, gS1, gS2]
    ssem = [sS0, sS1, sS2]
    isem = [iS0, iS1, iS2, iS3]
    dsem = [dS0, dS1, dS2]
    c = lax.axis_index("c")
    s = lax.axis_index("s")
    wid = s * NC + c
    base = wid * EW

    pltpu.sync_copy(zeros_hbm.at[pl.ds(s * RPT_MP, RPT_MP)],
                    acc.at[pl.ds(s * RPT_MP, RPT_MP)])
    pltpu.sync_copy(src_hbm.at[pl.ds(base, CH)], sv0)
    for m in (1, 2, 3):
        pltpu.async_copy(src_hbm.at[pl.ds(base + m * CH, CH)], srcv[m], isem[m])
    for m in (0, 1, 2):
        pltpu.sync_copy(dst_hbm.at[pl.ds(base + m * CH, CH)], dstv[m])
    plsc.subcore_barrier()

    def w_scatter(m):
        q = m % 3
        pltpu.make_async_copy(rows[q], acc.at[dstv[q]], ssem[q]).wait()

    def f_dst(m):
        q = m % 3
        pltpu.async_copy(dst_hbm.at[pl.ds(base + m * CH, CH)], dstv[q], dsem[q])

    def w_gather(m):
        q = m % 3
        pltpu.make_async_copy(g_hbm.at[srcv[m % 4]], rows[q], gsem[q]).wait()

    def w_dst(m):
        q = m % 3
        pltpu.make_async_copy(dst_hbm.at[pl.ds(base, CH)], dstv[q], dsem[q]).wait()

    def f_scatter(m):
        q = m % 3
        pltpu.async_copy(rows[q], acc.at[dstv[q]], ssem[q], add=True)

    def f_src(m):
        q = m % 4
        pltpu.async_copy(src_hbm.at[pl.ds(base + m * CH, CH)], srcv[q], isem[q])

    def w_src(m):
        q = m % 4
        pltpu.make_async_copy(src_hbm.at[pl.ds(base, CH)], srcv[q], isem[q]).wait()

    def f_gather(m):
        q = m % 3
        pltpu.async_copy(g_hbm.at[srcv[m % 4]], rows[q], gsem[q])

    def step(j):
        if 3 <= j and j - 3 < CHUNKS:
            w_scatter(j - 3)
        if 3 <= j < CHUNKS:
            f_dst(j)
        if 2 <= j and j - 2 < CHUNKS:
            w_gather(j - 2)
            if j >= 5:
                w_dst(j - 2)
            f_scatter(j - 2)
        if 2 <= j and j + 2 < CHUNKS:
            f_src(j + 2)
        if j < CHUNKS:
            if j >= 1:
                w_src(j)
            f_gather(j)

    for j in range(6):
        step(j)

    def steady(i, carry):
        for t in range(12):
            step_steady(6 + 12 * i + t, t)
        return carry

    def step_steady(j, t):
        jj = 6 + t
        q3, p3, r3 = jj % 3, (jj - 3) % 3, (jj - 2) % 3
        q4, p4 = jj % 4, (jj + 2) % 4
        pltpu.make_async_copy(rows[p3], acc.at[dstv[p3]], ssem[p3]).wait()
        pltpu.async_copy(dst_hbm.at[pl.ds(base + j * CH, CH)], dstv[p3], dsem[p3])
        pltpu.make_async_copy(g_hbm.at[srcv[(jj - 2) % 4]], rows[r3], gsem[r3]).wait()
        pltpu.make_async_copy(dst_hbm.at[pl.ds(base, CH)], dstv[r3], dsem[r3]).wait()
        pltpu.async_copy(rows[r3], acc.at[dstv[r3]], ssem[r3], add=True)
        pltpu.async_copy(src_hbm.at[pl.ds(base + (j + 2) * CH, CH)], srcv[p4], isem[p4])
        pltpu.make_async_copy(src_hbm.at[pl.ds(base, CH)], srcv[q4], isem[q4]).wait()
        pltpu.async_copy(g_hbm.at[srcv[q4]], rows[q3], gsem[q3])

    lax.fori_loop(0, (CHUNKS - 8) // 12, steady, 0)

    for j in range(CHUNKS - 2, CHUNKS + 3):
        step(j)

    plsc.subcore_barrier()
    pltpu.sync_copy(acc.at[pl.ds(s * RPT_MP, RPT_MP)],
                    out_hbm.at[c, pl.ds(s * RPT_MP, RPT_MP)])


_mp_call = pl.kernel(
    _mp_body,
    out_type=jax.ShapeDtypeStruct((NC, N_MP, H), jnp.float32),
    mesh=_mesh,
    scratch_types=(
        [pltpu.VMEM((CH, H), jnp.float32)] * 3
        + [pltpu.VMEM((CH,), jnp.int32)] * 7
        + [pltpu.VMEM_SHARED((N_MP, H), jnp.float32)]
        + [pltpu.SemaphoreType.DMA] * 13
    ),
)


BN = 1024
GN = N_PAD // BN


def _dinv_of(deg_blk):
    d = deg_blk[:, 0:1] + deg_blk[:, 1:2]
    return lax.rsqrt(d + 1.0)


def _l1_body(deg_ref, x_ref, w1_ref, g1_ref):
    dinv = _dinv_of(deg_ref)
    g = lax.dot_general(x_ref[...], w1_ref[...], (((0,), (0,)), ((), ())),
                        preferred_element_type=jnp.float32)
    g1_ref[...] = g * dinv


_l1_call = pl.pallas_call(
    _l1_body,
    grid=(GN,),
    in_specs=[
        pl.BlockSpec((BN, NC), lambda i: (i, 0)),
        pl.BlockSpec((D, BN), lambda i: (0, i)),
        pl.BlockSpec((D, H), lambda i: (0, 0)),
    ],
    out_specs=pl.BlockSpec((BN, H), lambda i: (i, 0)),
    out_shape=jax.ShapeDtypeStruct((N_PAD, H), jnp.float32),
)


def _l2_body(deg_ref, s1_ref, g1_ref, w2_ref, b1_ref, g2_ref):
    dinv = _dinv_of(deg_ref)
    h1 = jnp.maximum(dinv * (s1_ref[0] + s1_ref[1] + g1_ref[...]) + b1_ref[...], 0.0)
    g2_ref[...] = dinv * jnp.dot(h1, w2_ref[...], preferred_element_type=jnp.float32)


_l2_call = pl.pallas_call(
    _l2_body,
    grid=(GN,),
    in_specs=[
        pl.BlockSpec((BN, NC), lambda i: (i, 0)),
        pl.BlockSpec((NC, BN, H), lambda i: (0, i, 0)),
        pl.BlockSpec((BN, H), lambda i: (i, 0)),
        pl.BlockSpec((H, H), lambda i: (0, 0)),
        pl.BlockSpec((1, H), lambda i: (0, 0)),
    ],
    out_specs=pl.BlockSpec((BN, H), lambda i: (i, 0)),
    out_shape=jax.ShapeDtypeStruct((N_PAD, H), jnp.float32),
)


def _l3_body(deg_ref, s2_ref, g2_ref, b2_ref, h2_ref):
    dinv = _dinv_of(deg_ref)
    h2_ref[...] = dinv * (s2_ref[0] + s2_ref[1] + g2_ref[...]) + b2_ref[...]


_l3_call = pl.pallas_call(
    _l3_body,
    grid=(GN,),
    in_specs=[
        pl.BlockSpec((BN, NC), lambda i: (i, 0)),
        pl.BlockSpec((NC, BN, H), lambda i: (0, i, 0)),
        pl.BlockSpec((BN, H), lambda i: (i, 0)),
        pl.BlockSpec((1, H), lambda i: (0, 0)),
    ],
    out_specs=pl.BlockSpec((BN, H), lambda i: (i, 0)),
    out_shape=jax.ShapeDtypeStruct((N_PAD, H), jnp.float32),
)


BB = 256


def _out_body(o_ref, h2_ref, out_ref):
    out_ref[...] = jnp.dot(o_ref[...], h2_ref[...], preferred_element_type=jnp.float32)


_out_call = pl.pallas_call(
    _out_body,
    grid=(B // BB,),
    in_specs=[
        pl.BlockSpec((BB, N), lambda i: (i, 0)),
        pl.BlockSpec((N, A), lambda i: (0, 0)),
    ],
    out_specs=pl.BlockSpec((BB, A), lambda i: (i, 0)),
    out_shape=jax.ShapeDtypeStruct((B, A), jnp.float32),
)


def kernel(x, edge_index, onehot_values, W1, b1, W2, b2):
    ei = edge_index.astype(jnp.int32)
    pad = N + (jnp.arange(E_PAD - E, dtype=jnp.int32) % (N_MP - N))
    src = jnp.concatenate([ei[0], pad])
    dst = jnp.concatenate([ei[1], pad])
    xp = jnp.pad(x, ((0, 0), (0, N_PAD - N)))
    w2p = jnp.pad(W2, ((0, 0), (0, H - A)))
    b1r = b1.reshape(1, H)
    b2r = jnp.pad(b2, (0, H - A)).reshape(1, H)
    zeros_1 = jnp.zeros((N_PAD,), jnp.float32)
    zeros_c = jnp.zeros((N_PAD, H), jnp.float32)

    degp = _deg_call(dst, zeros_1)
    deg = degp.reshape(NC, N_PAD).T
    g1 = _l1_call(deg, xp, W1)
    s1 = _mp_call(g1, src, dst, zeros_c)
    g2 = _l2_call(deg, s1, g1, w2p, b1r)
    s2 = _mp_call(g2, src, dst, zeros_c)
    h2 = _l3_call(deg, s2, g2, b2r)
    return _out_call(onehot_values, h2[:N, :A])


# speedup vs baseline: 3.9521x; 1.0117x over previous
import jax
import jax.numpy as jnp
from jax import lax
from jax.experimental import pallas as pl
from jax.experimental.pallas import tpu as pltpu
from jax.experimental.pallas import tpu_sc as plsc

N = 10000
D = 128
H = 128
A = 64
B = 1024
E = 320000

NC, NS = 2, 16
NW = NC * NS
N_PAD = 10240
CH = 128
CHUNKS = 80
EW = CHUNKS * CH
E_PAD = NW * EW
RPT = N_PAD // NS
N_MP = 10112
RPT_MP = N_MP // NS
DEGW = 8

_mesh = plsc.VectorSubcoreMesh(core_axis_name="c", subcore_axis_name="s")


def _deg_body(dst_hbm, zeros_hbm, out_hbm, iv0, iv1, iv2, iv3, ones_v, acc,
              sS0, sS1, sS2, sS3, iS0, iS1, iS2, iS3):
    idxv = [iv0, iv1, iv2, iv3]
    ssem = [sS0, sS1, sS2, sS3]
    isem = [iS0, iS1, iS2, iS3]
    c = lax.axis_index("c")
    s = lax.axis_index("s")
    wid = s * NC + c
    base = wid * EW
    pltpu.sync_copy(zeros_hbm.at[pl.ds(s * RPT, RPT)], acc.at[pl.ds(s * RPT, RPT)])
    for k in range(CH // 16):
        ones_v[pl.ds(k * 16, 16)] = jnp.full((16,), 1.0, jnp.float32)
    pltpu.sync_copy(dst_hbm.at[pl.ds(base, CH)], iv0)
    for m in (1, 2, 3):
        pltpu.async_copy(dst_hbm.at[pl.ds(base + m * CH, CH)], idxv[m], isem[m])
    plsc.subcore_barrier()

    def step(j, static_q, steady_mode=False):
        q = static_q % 4
        if steady_mode or j >= 2:
            qq = (static_q - 2) % 4
            pltpu.make_async_copy(ones_v, acc.at[idxv[qq]], ssem[qq]).wait()
            if steady_mode or j + 2 < CHUNKS:
                pltpu.async_copy(dst_hbm.at[pl.ds(base + (j + 2) * CH, CH)],
                                 idxv[qq], isem[qq])
        if steady_mode or j < CHUNKS:
            if steady_mode or j >= 1:
                pltpu.make_async_copy(dst_hbm.at[pl.ds(base, CH)], idxv[q],
                                      isem[q]).wait()
            pltpu.async_copy(ones_v, acc.at[idxv[q]], ssem[q], add=True)

    for j in range(4):
        step(j, j)

    def steady(i, carry):
        for t in range(4):
            step(4 + 4 * i + t, t, steady_mode=True)
        return carry

    lax.fori_loop(0, (CHUNKS - 8) // 4, steady, 0)

    for j in range(CHUNKS - 4, CHUNKS + 2):
        step(j, j)

    plsc.subcore_barrier()
    pltpu.sync_copy(acc.at[pl.ds(s * RPT, RPT)],
                    out_hbm.at[pl.ds(c * N_PAD + s * RPT, RPT)])


_deg_call = pl.kernel(
    _deg_body,
    out_type=jax.ShapeDtypeStruct((NC * N_PAD,), jnp.float32),
    mesh=_mesh,
    scratch_types=(
        [pltpu.VMEM((CH,), jnp.int32)] * 4
        + [pltpu.VMEM((CH,), jnp.float32)]
        + [pltpu.VMEM_SHARED((N_PAD,), jnp.float32)]
        + [pltpu.SemaphoreType.DMA] * 8
    ),
)


def _mp_body(g_hbm, src_hbm, dst_hbm, zeros_hbm, out_hbm,
             r0, r1, r2, sv0, sv1, sv2, sv3, dv0, dv1, dv2, acc,
             gS0, gS1, gS2, sS0, sS1, sS2, iS0, iS1, iS2, iS3, dS0, dS1, dS2):
    rows = [r0, r1, r2]
    srcv = [sv0, sv1, sv2, sv3]
    dstv = [dv0, dv1, dv2]
    gsem = [gS0, gS1, gS2]
    ssem = [sS0, sS1, sS2]
    isem = [iS0, iS1, iS2, iS3]
    dsem = [dS0, dS1, dS2]
    c = lax.axis_index("c")
    s = lax.axis_index("s")
    wid = s * NC + c
    base = wid * EW

    pltpu.sync_copy(zeros_hbm.at[pl.ds(s * RPT_MP, RPT_MP)],
                    acc.at[pl.ds(s * RPT_MP, RPT_MP)])
    pltpu.sync_copy(src_hbm.at[pl.ds(base, CH)], sv0)
    for m in (1, 2, 3):
        pltpu.async_copy(src_hbm.at[pl.ds(base + m * CH, CH)], srcv[m], isem[m])
    for m in (0, 1, 2):
        pltpu.sync_copy(dst_hbm.at[pl.ds(base + m * CH, CH)], dstv[m])
    plsc.subcore_barrier()

    def w_scatter(m):
        q = m % 3
        pltpu.make_async_copy(rows[q], acc.at[dstv[q]], ssem[q]).wait()

    def f_dst(m):
        q = m % 3
        pltpu.async_copy(dst_hbm.at[pl.ds(base + m * CH, CH)], dstv[q], dsem[q])

    def w_gather(m):
        q = m % 3
        pltpu.make_async_copy(g_hbm.at[srcv[m % 4]], rows[q], gsem[q]).wait()

    def w_dst(m):
        q = m % 3
        pltpu.make_async_copy(dst_hbm.at[pl.ds(base, CH)], dstv[q], dsem[q]).wait()

    def f_scatter(m):
        q = m % 3
        pltpu.async_copy(rows[q], acc.at[dstv[q]], ssem[q], add=True)

    def f_src(m):
        q = m % 4
        pltpu.async_copy(src_hbm.at[pl.ds(base + m * CH, CH)], srcv[q], isem[q])

    def w_src(m):
        q = m % 4
        pltpu.make_async_copy(src_hbm.at[pl.ds(base, CH)], srcv[q], isem[q]).wait()

    def f_gather(m):
        q = m % 3
        pltpu.async_copy(g_hbm.at[srcv[m % 4]], rows[q], gsem[q])

    def step(j):
        if 3 <= j and j - 3 < CHUNKS:
            w_scatter(j - 3)
        if 3 <= j < CHUNKS:
            f_dst(j)
        if 2 <= j and j - 2 < CHUNKS:
            w_gather(j - 2)
            if j >= 5:
                w_dst(j - 2)
            f_scatter(j - 2)
        if 2 <= j and j + 2 < CHUNKS:
            f_src(j + 2)
        if j < CHUNKS:
            if j >= 1:
                w_src(j)
            f_gather(j)

    for j in range(6):
        step(j)

    def steady(i, carry):
        for t in range(12):
            step_steady(6 + 12 * i + t, t)
        return carry

    def step_steady(j, t):
        jj = 6 + t
        q3, p3, r3 = jj % 3, (jj - 3) % 3, (jj - 2) % 3
        q4, p4 = jj % 4, (jj + 2) % 4
        pltpu.make_async_copy(rows[p3], acc.at[dstv[p3]], ssem[p3]).wait()
        pltpu.async_copy(dst_hbm.at[pl.ds(base + j * CH, CH)], dstv[p3], dsem[p3])
        pltpu.make_async_copy(g_hbm.at[srcv[(jj - 2) % 4]], rows[r3], gsem[r3]).wait()
        pltpu.make_async_copy(dst_hbm.at[pl.ds(base, CH)], dstv[r3], dsem[r3]).wait()
        pltpu.async_copy(rows[r3], acc.at[dstv[r3]], ssem[r3], add=True)
        pltpu.async_copy(src_hbm.at[pl.ds(base + (j + 2) * CH, CH)], srcv[p4], isem[p4])
        pltpu.make_async_copy(src_hbm.at[pl.ds(base, CH)], srcv[q4], isem[q4]).wait()
        pltpu.async_copy(g_hbm.at[srcv[q4]], rows[q3], gsem[q3])

    lax.fori_loop(0, (CHUNKS - 8) // 12, steady, 0)

    for j in range(CHUNKS - 2, CHUNKS + 3):
        step(j)

    plsc.subcore_barrier()
    pltpu.sync_copy(acc.at[pl.ds(s * RPT_MP, RPT_MP)],
                    out_hbm.at[c, pl.ds(s * RPT_MP, RPT_MP)])


_mp_call = pl.kernel(
    _mp_body,
    out_type=jax.ShapeDtypeStruct((NC, N_MP, H), jnp.float32),
    mesh=_mesh,
    scratch_types=(
        [pltpu.VMEM((CH, H), jnp.float32)] * 3
        + [pltpu.VMEM((CH,), jnp.int32)] * 7
        + [pltpu.VMEM_SHARED((N_MP, H), jnp.float32)]
        + [pltpu.SemaphoreType.DMA] * 13
    ),
)


BN = 1024
GN = N_PAD // BN


def _dinv_of(deg_blk):
    d = deg_blk[:, 0:1] + deg_blk[:, 1:2]
    return lax.rsqrt(d + 1.0)


def _l1_body(deg_ref, x_ref, w1_ref, g1_ref):
    dinv = _dinv_of(deg_ref)
    g = lax.dot_general(x_ref[...], w1_ref[...], (((0,), (0,)), ((), ())),
                        preferred_element_type=jnp.float32)
    g1_ref[...] = g * dinv


_l1_call = pl.pallas_call(
    _l1_body,
    grid=(GN,),
    in_specs=[
        pl.BlockSpec((BN, NC), lambda i: (i, 0)),
        pl.BlockSpec((D, BN), lambda i: (0, i)),
        pl.BlockSpec((D, H), lambda i: (0, 0)),
    ],
    out_specs=pl.BlockSpec((BN, H), lambda i: (i, 0)),
    out_shape=jax.ShapeDtypeStruct((N_PAD, H), jnp.float32),
)


def _l2_body(deg_ref, s1_ref, g1_ref, w2_ref, b1_ref, g2_ref):
    dinv = _dinv_of(deg_ref)
    h1 = jnp.maximum(dinv * (s1_ref[0] + s1_ref[1] + g1_ref[...]) + b1_ref[...], 0.0)
    g2_ref[...] = dinv * jnp.dot(h1, w2_ref[...], preferred_element_type=jnp.float32)


_l2_call = pl.pallas_call(
    _l2_body,
    grid=(GN,),
    in_specs=[
        pl.BlockSpec((BN, NC), lambda i: (i, 0)),
        pl.BlockSpec((NC, BN, H), lambda i: (0, i, 0)),
        pl.BlockSpec((BN, H), lambda i: (i, 0)),
        pl.BlockSpec((H, H), lambda i: (0, 0)),
        pl.BlockSpec((1, H), lambda i: (0, 0)),
    ],
    out_specs=pl.BlockSpec((BN, H), lambda i: (i, 0)),
    out_shape=jax.ShapeDtypeStruct((N_PAD, H), jnp.float32),
)


def _l3_body(deg_ref, s2_ref, g2_ref, b2_ref, h2_ref):
    dinv = _dinv_of(deg_ref)
    h2 = dinv * (s2_ref[0] + s2_ref[1] + g2_ref[...]) + b2_ref[...]
    h2_ref[...] = h2[:, :A]


_l3_call = pl.pallas_call(
    _l3_body,
    grid=(GN,),
    in_specs=[
        pl.BlockSpec((BN, NC), lambda i: (i, 0)),
        pl.BlockSpec((NC, BN, H), lambda i: (0, i, 0)),
        pl.BlockSpec((BN, H), lambda i: (i, 0)),
        pl.BlockSpec((1, H), lambda i: (0, 0)),
    ],
    out_specs=pl.BlockSpec((BN, A), lambda i: (i, 0)),
    out_shape=jax.ShapeDtypeStruct((N, A), jnp.float32),
)


BB = 256


def _out_body(o_ref, h2_ref, out_ref):
    out_ref[...] = jnp.dot(o_ref[...], h2_ref[...], preferred_element_type=jnp.float32)


_out_call = pl.pallas_call(
    _out_body,
    grid=(B // BB,),
    in_specs=[
        pl.BlockSpec((BB, N), lambda i: (i, 0)),
        pl.BlockSpec((N, A), lambda i: (0, 0)),
    ],
    out_specs=pl.BlockSpec((BB, A), lambda i: (i, 0)),
    out_shape=jax.ShapeDtypeStruct((B, A), jnp.float32),
)


def kernel(x, edge_index, onehot_values, W1, b1, W2, b2):
    ei = edge_index.astype(jnp.int32)
    pad = N + (jnp.arange(E_PAD - E, dtype=jnp.int32) % (N_MP - N))
    src = jnp.concatenate([ei[0], pad])
    dst = jnp.concatenate([ei[1], pad])
    xp = jnp.pad(x, ((0, 0), (0, N_PAD - N)))
    w2p = jnp.pad(W2, ((0, 0), (0, H - A)))
    b1r = b1.reshape(1, H)
    b2r = jnp.pad(b2, (0, H - A)).reshape(1, H)
    zeros_1 = jnp.zeros((N_PAD,), jnp.float32)
    zeros_c = jnp.zeros((N_PAD, H), jnp.float32)

    degp = _deg_call(dst, zeros_1)
    deg = degp.reshape(NC, N_PAD).T
    g1 = _l1_call(deg, xp, W1)
    s1 = _mp_call(g1, src, dst, zeros_c)
    g2 = _l2_call(deg, s1, g1, w2p, b1r)
    s2 = _mp_call(g2, src, dst, zeros_c)
    h2 = _l3_call(deg, s2, g2, b2r)
    return _out_call(onehot_values, h2)


# speedup vs baseline: 3.9543x; 1.0006x over previous
import jax
import jax.numpy as jnp
from jax import lax
from jax.experimental import pallas as pl
from jax.experimental.pallas import tpu as pltpu
from jax.experimental.pallas import tpu_sc as plsc

N = 10000
D = 128
H = 128
A = 64
B = 1024
E = 320000

NC, NS = 2, 16
NW = NC * NS
N_PAD = 10240
CH = 128
CHUNKS = 80
EW = CHUNKS * CH
E_PAD = NW * EW
RPT = N_PAD // NS
N_MP = 10112
RPT_MP = N_MP // NS
DEGW = 8

_mesh = plsc.VectorSubcoreMesh(core_axis_name="c", subcore_axis_name="s")


def _deg_body(dst_hbm, zeros_hbm, out_hbm, iv0, iv1, iv2, iv3, ones_v, acc,
              sS0, sS1, sS2, sS3, iS0, iS1, iS2, iS3):
    idxv = [iv0, iv1, iv2, iv3]
    ssem = [sS0, sS1, sS2, sS3]
    isem = [iS0, iS1, iS2, iS3]
    c = lax.axis_index("c")
    s = lax.axis_index("s")
    wid = s * NC + c
    base = wid * EW
    pltpu.sync_copy(zeros_hbm.at[pl.ds(s * RPT, RPT)], acc.at[pl.ds(s * RPT, RPT)])
    for k in range(CH // 16):
        ones_v[pl.ds(k * 16, 16)] = jnp.full((16,), 1.0, jnp.float32)
    pltpu.sync_copy(dst_hbm.at[pl.ds(base, CH)], iv0)
    for m in (1, 2, 3):
        pltpu.async_copy(dst_hbm.at[pl.ds(base + m * CH, CH)], idxv[m], isem[m])
    plsc.subcore_barrier()

    def step(j, static_q, steady_mode=False):
        q = static_q % 4
        if steady_mode or j >= 2:
            qq = (static_q - 2) % 4
            pltpu.make_async_copy(ones_v, acc.at[idxv[qq]], ssem[qq]).wait()
            if steady_mode or j + 2 < CHUNKS:
                pltpu.async_copy(dst_hbm.at[pl.ds(base + (j + 2) * CH, CH)],
                                 idxv[qq], isem[qq])
        if steady_mode or j < CHUNKS:
            if steady_mode or j >= 1:
                pltpu.make_async_copy(dst_hbm.at[pl.ds(base, CH)], idxv[q],
                                      isem[q]).wait()
            pltpu.async_copy(ones_v, acc.at[idxv[q]], ssem[q], add=True)

    for j in range(4):
        step(j, j)

    def steady(i, carry):
        for t in range(4):
            step(4 + 4 * i + t, t, steady_mode=True)
        return carry

    lax.fori_loop(0, (CHUNKS - 8) // 4, steady, 0)

    for j in range(CHUNKS - 4, CHUNKS + 2):
        step(j, j)

    plsc.subcore_barrier()
    pltpu.sync_copy(acc.at[pl.ds(s * RPT, RPT)],
                    out_hbm.at[pl.ds(c * N_PAD + s * RPT, RPT)])


_deg_call = pl.kernel(
    _deg_body,
    out_type=jax.ShapeDtypeStruct((NC * N_PAD,), jnp.float32),
    mesh=_mesh,
    scratch_types=(
        [pltpu.VMEM((CH,), jnp.int32)] * 4
        + [pltpu.VMEM((CH,), jnp.float32)]
        + [pltpu.VMEM_SHARED((N_PAD,), jnp.float32)]
        + [pltpu.SemaphoreType.DMA] * 8
    ),
)


def _mp_body(g_hbm, src_hbm, dst_hbm, zeros_hbm, out_hbm,
             r0, r1, r2, sv0, sv1, sv2, sv3, dv0, dv1, dv2, acc,
             gS0, gS1, gS2, sS0, sS1, sS2, iS0, iS1, iS2, iS3, dS0, dS1, dS2):
    rows = [r0, r1, r2]
    srcv = [sv0, sv1, sv2, sv3]
    dstv = [dv0, dv1, dv2]
    gsem = [gS0, gS1, gS2]
    ssem = [sS0, sS1, sS2]
    isem = [iS0, iS1, iS2, iS3]
    dsem = [dS0, dS1, dS2]
    c = lax.axis_index("c")
    s = lax.axis_index("s")
    wid = s * NC + c
    base = wid * EW

    pltpu.sync_copy(zeros_hbm.at[pl.ds(s * RPT_MP, RPT_MP)],
                    acc.at[pl.ds(s * RPT_MP, RPT_MP)])
    pltpu.sync_copy(src_hbm.at[pl.ds(base, CH)], sv0)
    for m in (1, 2, 3):
        pltpu.async_copy(src_hbm.at[pl.ds(base + m * CH, CH)], srcv[m], isem[m])
    for m in (0, 1, 2):
        pltpu.sync_copy(dst_hbm.at[pl.ds(base + m * CH, CH)], dstv[m])
    plsc.subcore_barrier()

    def w_scatter(m):
        q = m % 3
        pltpu.make_async_copy(rows[q], acc.at[dstv[q]], ssem[q]).wait()

    def f_dst(m):
        q = m % 3
        pltpu.async_copy(dst_hbm.at[pl.ds(base + m * CH, CH)], dstv[q], dsem[q])

    def w_gather(m):
        q = m % 3
        pltpu.make_async_copy(g_hbm.at[srcv[m % 4]], rows[q], gsem[q]).wait()

    def w_dst(m):
        q = m % 3
        pltpu.make_async_copy(dst_hbm.at[pl.ds(base, CH)], dstv[q], dsem[q]).wait()

    def f_scatter(m):
        q = m % 3
        pltpu.async_copy(rows[q], acc.at[dstv[q]], ssem[q], add=True)

    def f_src(m):
        q = m % 4
        pltpu.async_copy(src_hbm.at[pl.ds(base + m * CH, CH)], srcv[q], isem[q])

    def w_src(m):
        q = m % 4
        pltpu.make_async_copy(src_hbm.at[pl.ds(base, CH)], srcv[q], isem[q]).wait()

    def f_gather(m):
        q = m % 3
        pltpu.async_copy(g_hbm.at[srcv[m % 4]], rows[q], gsem[q])

    def step(j):
        if 3 <= j and j - 3 < CHUNKS:
            w_scatter(j - 3)
        if 3 <= j < CHUNKS:
            f_dst(j)
        if 2 <= j and j - 2 < CHUNKS:
            w_gather(j - 2)
            if j >= 5:
                w_dst(j - 2)
            f_scatter(j - 2)
        if 2 <= j and j + 2 < CHUNKS:
            f_src(j + 2)
        if j < CHUNKS:
            if j >= 1:
                w_src(j)
            f_gather(j)

    for j in range(6):
        step(j)

    def steady(i, carry):
        for t in range(12):
            step_steady(6 + 12 * i + t, t)
        return carry

    def step_steady(j, t):
        jj = 6 + t
        q3, p3, r3 = jj % 3, (jj - 3) % 3, (jj - 2) % 3
        q4, p4 = jj % 4, (jj + 2) % 4
        pltpu.make_async_copy(rows[p3], acc.at[dstv[p3]], ssem[p3]).wait()
        pltpu.async_copy(dst_hbm.at[pl.ds(base + j * CH, CH)], dstv[p3], dsem[p3])
        pltpu.make_async_copy(g_hbm.at[srcv[(jj - 2) % 4]], rows[r3], gsem[r3]).wait()
        pltpu.make_async_copy(dst_hbm.at[pl.ds(base, CH)], dstv[r3], dsem[r3]).wait()
        pltpu.async_copy(rows[r3], acc.at[dstv[r3]], ssem[r3], add=True)
        pltpu.async_copy(src_hbm.at[pl.ds(base + (j + 2) * CH, CH)], srcv[p4], isem[p4])
        pltpu.make_async_copy(src_hbm.at[pl.ds(base, CH)], srcv[q4], isem[q4]).wait()
        pltpu.async_copy(g_hbm.at[srcv[q4]], rows[q3], gsem[q3])

    lax.fori_loop(0, (CHUNKS - 8) // 12, steady, 0)

    for j in range(CHUNKS - 2, CHUNKS + 3):
        step(j)

    plsc.subcore_barrier()
    pltpu.sync_copy(acc.at[pl.ds(s * RPT_MP, RPT_MP)],
                    out_hbm.at[c, pl.ds(s * RPT_MP, RPT_MP)])


_mp_call = pl.kernel(
    _mp_body,
    out_type=jax.ShapeDtypeStruct((NC, N_MP, H), jnp.float32),
    mesh=_mesh,
    scratch_types=(
        [pltpu.VMEM((CH, H), jnp.float32)] * 3
        + [pltpu.VMEM((CH,), jnp.int32)] * 7
        + [pltpu.VMEM_SHARED((N_MP, H), jnp.float32)]
        + [pltpu.SemaphoreType.DMA] * 13
    ),
)


BN = 1024
GN = N_PAD // BN


def _dinv_of(deg_blk):
    d = deg_blk[:, 0:1] + deg_blk[:, 1:2]
    return lax.rsqrt(d + 1.0)


def _l1_body(deg_ref, x_ref, w1_ref, g1_ref):
    dinv = _dinv_of(deg_ref)
    g = lax.dot_general(x_ref[...], w1_ref[...], (((0,), (0,)), ((), ())),
                        preferred_element_type=jnp.float32)
    g1_ref[...] = g * dinv


_l1_call = pl.pallas_call(
    _l1_body,
    grid=(GN,),
    in_specs=[
        pl.BlockSpec((BN, NC), lambda i: (i, 0)),
        pl.BlockSpec((D, BN), lambda i: (0, i)),
        pl.BlockSpec((D, H), lambda i: (0, 0)),
    ],
    out_specs=pl.BlockSpec((BN, H), lambda i: (i, 0)),
    out_shape=jax.ShapeDtypeStruct((N_PAD, H), jnp.float32),
)


def _l2_body(deg_ref, s1_ref, g1_ref, w2_ref, b1_ref, g2_ref):
    dinv = _dinv_of(deg_ref)
    h1 = jnp.maximum(dinv * (s1_ref[0] + s1_ref[1] + g1_ref[...]) + b1_ref[...], 0.0)
    g2_ref[...] = dinv * jnp.dot(h1, w2_ref[...], preferred_element_type=jnp.float32)


_l2_call = pl.pallas_call(
    _l2_body,
    grid=(GN,),
    in_specs=[
        pl.BlockSpec((BN, NC), lambda i: (i, 0)),
        pl.BlockSpec((NC, BN, H), lambda i: (0, i, 0)),
        pl.BlockSpec((BN, H), lambda i: (i, 0)),
        pl.BlockSpec((H, H), lambda i: (0, 0)),
        pl.BlockSpec((1, H), lambda i: (0, 0)),
    ],
    out_specs=pl.BlockSpec((BN, H), lambda i: (i, 0)),
    out_shape=jax.ShapeDtypeStruct((N_PAD, H), jnp.float32),
)


def _l3_body(deg_ref, s2_ref, g2_ref, b2_ref, h2_ref):
    dinv = _dinv_of(deg_ref)
    h2 = dinv * (s2_ref[0] + s2_ref[1] + g2_ref[...]) + b2_ref[...]
    h2_ref[...] = h2[:, :A]


_l3_call = pl.pallas_call(
    _l3_body,
    grid=(GN,),
    in_specs=[
        pl.BlockSpec((BN, NC), lambda i: (i, 0)),
        pl.BlockSpec((NC, BN, H), lambda i: (0, i, 0)),
        pl.BlockSpec((BN, H), lambda i: (i, 0)),
        pl.BlockSpec((1, H), lambda i: (0, 0)),
    ],
    out_specs=pl.BlockSpec((BN, A), lambda i: (i, 0)),
    out_shape=jax.ShapeDtypeStruct((N, A), jnp.float32),
)


BB = 256


def _out_body(o_ref, h2_ref, out_ref):
    out_ref[...] = jnp.dot(o_ref[...], h2_ref[...], preferred_element_type=jnp.float32)


_out_call = pl.pallas_call(
    _out_body,
    grid=(B // BB,),
    in_specs=[
        pl.BlockSpec((BB, N), lambda i: (i, 0)),
        pl.BlockSpec((N, A), lambda i: (0, 0)),
    ],
    out_specs=pl.BlockSpec((BB, A), lambda i: (i, 0)),
    out_shape=jax.ShapeDtypeStruct((B, A), jnp.float32),
)


def kernel(x, edge_index, onehot_values, W1, b1, W2, b2):
    ei = edge_index.astype(jnp.int32)
    pad = N + (jnp.arange(E_PAD - E, dtype=jnp.int32) % (N_MP - N))
    src = jnp.concatenate([ei[0], pad])
    dst = jnp.concatenate([ei[1], pad])
    w2p = jnp.pad(W2, ((0, 0), (0, H - A)))
    b1r = b1.reshape(1, H)
    b2r = jnp.pad(b2, (0, H - A)).reshape(1, H)
    zeros_1 = jnp.zeros((N_PAD,), jnp.float32)
    zeros_c = jnp.zeros((N_PAD, H), jnp.float32)

    degp = _deg_call(dst, zeros_1)
    deg = degp.reshape(NC, N_PAD).T
    g1 = _l1_call(deg, x, W1)
    s1 = _mp_call(g1, src, dst, zeros_c)
    g2 = _l2_call(deg, s1, g1, w2p, b1r)
    s2 = _mp_call(g2, src, dst, zeros_c)
    h2 = _l3_call(deg, s2, g2, b2r)
    return _out_call(onehot_values, h2)
